# baseline (speedup 1.0000x reference)
import functools

import jax
import jax.numpy as jnp
from jax import lax
from jax.experimental import pallas as pl
from jax.experimental.pallas import tpu as pltpu
from jax.experimental.pallas import tpu_sc as plsc

HIDDEN = 1024
B_TOTAL = 4 * 8192
NUM_WORKERS = 32
B_PER_W = B_TOTAL // NUM_WORKERS
CHUNK = 8
NCHUNK = B_PER_W // CHUNK
NBUF = 8
NGROUP = NCHUNK // NBUF

_mesh = plsc.VectorSubcoreMesh(core_axis_name="c", subcore_axis_name="s")


@functools.partial(
    pl.kernel,
    out_type=jax.ShapeDtypeStruct((B_TOTAL, HIDDEN), jnp.float32),
    mesh=_mesh,
    scratch_types=[
        pltpu.VMEM((B_PER_W,), jnp.int32),
        [pltpu.VMEM((CHUNK, HIDDEN), jnp.float32)] * NBUF,
        [pltpu.SemaphoreType.DMA] * NBUF,
        [pltpu.SemaphoreType.DMA] * NBUF,
    ],
)
def _gather_rows(idx_hbm, table_hbm, out_hbm, idx_v, bufs, gsems, wsems):
    wid = lax.axis_index("s") * 2 + lax.axis_index("c")
    base = wid * B_PER_W
    pltpu.sync_copy(idx_hbm.at[pl.ds(base, B_PER_W)], idx_v)

    def g_start(c, j):
        pltpu.async_copy(table_hbm.at[idx_v.at[pl.ds(c * CHUNK, CHUNK)]],
                         bufs[j], gsems[j])

    def g_wait(j):
        pltpu.make_async_copy(table_hbm.at[idx_v.at[pl.ds(0, CHUNK)]],
                              bufs[j], gsems[j]).wait()

    def w_start(c, j):
        pltpu.async_copy(bufs[j], out_hbm.at[pl.ds(base + c * CHUNK, CHUNK)],
                         wsems[j])

    def w_wait(j):
        pltpu.make_async_copy(bufs[j], out_hbm.at[pl.ds(base, CHUNK)],
                              wsems[j]).wait()

    for j in range(NBUF):
        g_start(j, j)

    NG = NBUF // 2
    for j in range(NG):
        g_start(j, j)
        w_start(j, NG + j)

    def group(h, carry):
        c0 = h * NG
        for j in range(NG):
            g_wait(j)
            w_wait(NG + j)

        @pl.when(h < 2 * NGROUP - 1)
        def _():
            for j in range(NG):
                g_start(c0 + NG + j, j)
                w_start(c0 + NG + j, NG + j)

        return carry

    lax.fori_loop(0, 2 * NGROUP, group, 0)


def kernel(position_ids, pos_enc):
    idx = position_ids.reshape(B_TOTAL).astype(jnp.int32)
    out = _gather_rows(idx, pos_enc)
    return out.reshape(position_ids.shape + (HIDDEN,))


# speedup vs baseline: 1.0050x; 1.0050x over previous
import functools

import jax
import jax.numpy as jnp
from jax import lax
from jax.experimental import pallas as pl
from jax.experimental.pallas import tpu as pltpu
from jax.experimental.pallas import tpu_sc as plsc

HIDDEN = 1024
B_TOTAL = 4 * 8192
NUM_WORKERS = 32
B_PER_W = B_TOTAL // NUM_WORKERS
CHUNK = 8
NCHUNK = B_PER_W // CHUNK
NBUF = 8
NGROUP = NCHUNK // NBUF

_mesh = plsc.VectorSubcoreMesh(core_axis_name="c", subcore_axis_name="s")


@functools.partial(
    pl.kernel,
    out_type=jax.ShapeDtypeStruct((B_TOTAL, HIDDEN), jnp.float32),
    mesh=_mesh,
    scratch_types=[
        pltpu.VMEM((B_PER_W,), jnp.int32),
        [pltpu.VMEM((CHUNK, HIDDEN), jnp.float32)] * NBUF,
        [pltpu.SemaphoreType.DMA] * NBUF,
        [pltpu.SemaphoreType.DMA] * NBUF,
    ],
)
def _gather_rows(idx_hbm, table_hbm, out_hbm, idx_v, bufs, gsems, wsems):
    wid = lax.axis_index("s") * 2 + lax.axis_index("c")
    base = wid * B_PER_W
    pltpu.sync_copy(idx_hbm.at[pl.ds(base, B_PER_W)], idx_v)

    def g_start(c, j):
        pltpu.async_copy(table_hbm.at[idx_v.at[pl.ds(c * CHUNK, CHUNK)]],
                         bufs[j], gsems[j])

    def g_wait(j):
        pltpu.make_async_copy(table_hbm.at[idx_v.at[pl.ds(0, CHUNK)]],
                              bufs[j], gsems[j]).wait()

    def w_start(c, j):
        pltpu.async_copy(bufs[j], out_hbm.at[pl.ds(base + c * CHUNK, CHUNK)],
                         wsems[j])

    def w_wait(j):
        pltpu.make_async_copy(bufs[j], out_hbm.at[pl.ds(base, CHUNK)],
                              wsems[j]).wait()

    for j in range(NBUF):
        g_start(j, j)

    def group(h, carry):
        c0 = h * NBUF
        for j in range(NBUF):
            g_wait(j)
            w_start(c0 + j, j)

        @pl.when(h < NGROUP - 1)
        def _():
            for j in range(NBUF):
                w_wait(j)
                g_start(c0 + NBUF + j, j)

        return carry

    lax.fori_loop(0, NGROUP, group, 0)
    for j in range(NBUF):
        w_wait(j)


def kernel(position_ids, pos_enc):
    idx = position_ids.reshape(B_TOTAL).astype(jnp.int32)
    out = _gather_rows(idx, pos_enc)
    return out.reshape(position_ids.shape + (HIDDEN,))
